# baseline (speedup 1.0000x reference)
import functools

import jax
import jax.numpy as jnp
from jax.experimental import pallas as pl
from jax.experimental.pallas import tpu as pltpu

B, S, H = 4, 2048, 2048
E, K, D = 16, 2, 64
N = B * S
TB = 512


def _router_body(x_ref, wcat_ref, be1_ref, w2t_ref, be2_ref,
                 wc1t_ref, bc1_ref, wc2t_ref, bc2_ref, ee_ref, wb0_ref,
                 bb_ref, scores_ref, ts_ref, ti_ref):
    bf = jnp.bfloat16
    dot = lambda a, b: jnp.dot(a.astype(bf), b.astype(bf),
                               preferred_element_type=jnp.float32)
    dot_t = lambda a, b: jax.lax.dot_general(
        a.astype(bf), b.astype(bf), (((1,), (1,)), ((), ())),
        preferred_element_type=jnp.float32)

    xb = x_ref[...]
    fused = dot(xb, wcat_ref[...])
    h1 = fused[:, 0:D] + be1_ref[...]
    h1 = h1 * jax.nn.sigmoid(h1)
    emb = dot(h1, w2t_ref[...]) + be2_ref[...]

    ec = dot(ee_ref[...], wc1t_ref[...]) + bc1_ref[...]
    ec = ec * jax.nn.sigmoid(ec)
    enc = dot(ec, wc2t_ref[...]) + bc2_ref[...]

    p = dot_t(enc, wb0_ref[...])
    simT = dot_t(p, emb) + bb_ref[0, 0]

    logitsT = jnp.transpose(fused[:, D:D + E])
    combT = logitsT + 0.3 * simT
    m = jnp.max(combT, axis=0, keepdims=True)
    ex = jnp.exp(combT - m)
    scT = ex / jnp.sum(ex, axis=0, keepdims=True)
    scores_ref[...] = jnp.transpose(scT)

    idx = jax.lax.broadcasted_iota(jnp.int32, scT.shape, 0)
    s1 = jnp.max(scT, axis=0, keepdims=True)
    i1 = jnp.min(jnp.where(scT == s1, idx, E), axis=0, keepdims=True)
    masked = jnp.where(idx == i1, -1.0, scT)
    s2 = jnp.max(masked, axis=0, keepdims=True)
    i2 = jnp.min(jnp.where(masked == s2, idx, E), axis=0, keepdims=True)

    t = jnp.exp(s2 - s1)
    p1 = 1.0 / (1.0 + t)
    io = jax.lax.broadcasted_iota(jnp.int32, (K, s1.shape[1]), 0)
    ts_ref[...] = jnp.transpose(jnp.where(io == 0, p1, t * p1))
    ti_ref[...] = jnp.transpose(jnp.where(io == 0, i1, i2))


@functools.partial(jax.jit, static_argnames=("interpret",))
def _router(xf, wcat, be1, w2t, be2, wc1t, bc1, wc2t, bc2, ee, wb0, bb2,
            interpret=False):
    grid = (N // TB,)
    full = lambda shape: pl.BlockSpec(shape, lambda i: (0,) * len(shape))
    return pl.pallas_call(
        _router_body,
        grid=grid,
        in_specs=[
            pl.BlockSpec((TB, H), lambda i: (i, 0)),
            full((H, 128)), full((1, D)), full((D, D)),
            full((1, D)), full((D, D)), full((1, D)), full((D, D)),
            full((1, D)), full((E, D)), full((D, D)), full((1, 1)),
        ],
        out_specs=[
            pl.BlockSpec((TB, E), lambda i: (i, 0)),
            pl.BlockSpec((TB, K), lambda i: (i, 0)),
            pl.BlockSpec((TB, K), lambda i: (i, 0)),
        ],
        out_shape=[
            jax.ShapeDtypeStruct((N, E), jnp.float32),
            jax.ShapeDtypeStruct((N, K), jnp.float32),
            jax.ShapeDtypeStruct((N, K), jnp.int32),
        ],
        compiler_params=pltpu.CompilerParams(
            dimension_semantics=("parallel",)),
        interpret=interpret,
    )(xf, wcat, be1, w2t, be2, wc1t, bc1, wc2t, bc2, ee, wb0, bb2)


def kernel(x, gate_W, We1, be1, We2, be2, Wc1, bc1, Wc2, bc2,
           expert_embeddings, Wb, bb, interpret=False):
    xf = x.reshape(-1, H)
    wcat = jnp.concatenate(
        [We1.T, gate_W.T, jnp.zeros((H, 128 - D - E), jnp.float32)], axis=1)
    scores, ts, ti = _router(
        xf, wcat, be1.reshape(1, D), We2.T, be2.reshape(1, D),
        Wc1.T, bc1.reshape(1, D), Wc2.T, bc2.reshape(1, D),
        expert_embeddings, Wb[0], bb.reshape(1, 1), interpret=interpret)
    return (ts, ti, scores)


# speedup vs baseline: 1.5499x; 1.5499x over previous
import functools

import jax
import jax.numpy as jnp
from jax.experimental import pallas as pl
from jax.experimental.pallas import tpu as pltpu

B, S, H = 4, 2048, 2048
E, K, D = 16, 2, 64
N = B * S
TB = 1024


def _router_body(x_ref, wcat_ref, be1_ref, w2t_ref, be2_ref,
                 wc1t_ref, bc1_ref, wc2t_ref, bc2_ref, ee_ref, wb0_ref,
                 bb_ref, scores_ref, ts_ref, ti_ref):
    bf = jnp.bfloat16
    dot = lambda a, b: jnp.dot(a.astype(bf), b.astype(bf),
                               preferred_element_type=jnp.float32)
    dot_t = lambda a, b: jax.lax.dot_general(
        a.astype(bf), b.astype(bf), (((1,), (1,)), ((), ())),
        preferred_element_type=jnp.float32)

    xb = x_ref[...]
    fused = dot(xb, wcat_ref[...])
    h1 = fused[:, 0:D] + be1_ref[...]
    h1 = h1 * jax.nn.sigmoid(h1)
    emb = dot(h1, w2t_ref[...]) + be2_ref[...]

    ec = dot(ee_ref[...], wc1t_ref[...]) + bc1_ref[...]
    ec = ec * jax.nn.sigmoid(ec)
    enc = dot(ec, wc2t_ref[...]) + bc2_ref[...]

    p = dot_t(enc, wb0_ref[...])
    simT = dot_t(p, emb) + bb_ref[0, 0]

    logitsT = jnp.transpose(fused[:, D:D + E])
    combT = logitsT + 0.3 * simT
    m = jnp.max(combT, axis=0, keepdims=True)
    ex = jnp.exp(combT - m)
    scT = ex / jnp.sum(ex, axis=0, keepdims=True)
    scores_ref[...] = scT

    idx = jax.lax.broadcasted_iota(jnp.int32, scT.shape, 0)
    s1 = jnp.max(scT, axis=0, keepdims=True)
    i1 = jnp.min(jnp.where(scT == s1, idx, E), axis=0, keepdims=True)
    masked = jnp.where(idx == i1, -1.0, scT)
    s2 = jnp.max(masked, axis=0, keepdims=True)
    i2 = jnp.min(jnp.where(masked == s2, idx, E), axis=0, keepdims=True)

    t = jnp.exp(s2 - s1)
    p1 = 1.0 / (1.0 + t)
    io = jax.lax.broadcasted_iota(jnp.int32, (K, s1.shape[1]), 0)
    ts_ref[...] = jnp.where(io == 0, p1, t * p1)
    ti_ref[...] = jnp.where(io == 0, i1, i2)


@functools.partial(jax.jit, static_argnames=("interpret",))
def _router(xf, wcat, be1, w2t, be2, wc1t, bc1, wc2t, bc2, ee, wb0, bb2,
            interpret=False):
    grid = (N // TB,)
    full = lambda shape: pl.BlockSpec(shape, lambda i: (0,) * len(shape))
    return pl.pallas_call(
        _router_body,
        grid=grid,
        in_specs=[
            pl.BlockSpec((TB, H), lambda i: (i, 0)),
            full((H, 128)), full((1, D)), full((D, D)),
            full((1, D)), full((D, D)), full((1, D)), full((D, D)),
            full((1, D)), full((E, D)), full((D, D)), full((1, 1)),
        ],
        out_specs=[
            pl.BlockSpec((E, TB), lambda i: (0, i)),
            pl.BlockSpec((K, TB), lambda i: (0, i)),
            pl.BlockSpec((K, TB), lambda i: (0, i)),
        ],
        out_shape=[
            jax.ShapeDtypeStruct((E, N), jnp.float32),
            jax.ShapeDtypeStruct((K, N), jnp.float32),
            jax.ShapeDtypeStruct((K, N), jnp.int32),
        ],
        compiler_params=pltpu.CompilerParams(
            dimension_semantics=("parallel",)),
        interpret=interpret,
    )(xf, wcat, be1, w2t, be2, wc1t, bc1, wc2t, bc2, ee, wb0, bb2)


def kernel(x, gate_W, We1, be1, We2, be2, Wc1, bc1, Wc2, bc2,
           expert_embeddings, Wb, bb, interpret=False):
    xf = x.reshape(-1, H)
    wcat = jnp.concatenate(
        [We1.T, gate_W.T, jnp.zeros((H, 128 - D - E), jnp.float32)], axis=1)
    scoresT, tsT, tiT = _router(
        xf, wcat, be1.reshape(1, D), We2.T, be2.reshape(1, D),
        Wc1.T, bc1.reshape(1, D), Wc2.T, bc2.reshape(1, D),
        expert_embeddings, Wb[0], bb.reshape(1, 1), interpret=interpret)
    return (tsT.T, tiT.T, scoresT.T)


# speedup vs baseline: 1.5626x; 1.0082x over previous
import functools

import jax
import jax.numpy as jnp
from jax.experimental import pallas as pl
from jax.experimental.pallas import tpu as pltpu

B, S, H = 4, 2048, 2048
E, K, D = 16, 2, 64
N = B * S
TB = 2048


def _router_body(x_ref, wcat_ref, be1_ref, w2t_ref, be2_ref,
                 wc1t_ref, bc1_ref, wc2t_ref, bc2_ref, ee_ref, wb0_ref,
                 bb_ref, scores_ref, ts_ref, ti_ref):
    bf = jnp.bfloat16
    dot = lambda a, b: jnp.dot(a.astype(bf), b.astype(bf),
                               preferred_element_type=jnp.float32)
    dot_t = lambda a, b: jax.lax.dot_general(
        a.astype(bf), b.astype(bf), (((1,), (1,)), ((), ())),
        preferred_element_type=jnp.float32)

    xb = x_ref[...]
    fused = dot(xb, wcat_ref[...])
    h1 = fused[:, 0:D] + be1_ref[...]
    h1 = h1 * jax.nn.sigmoid(h1)
    emb = dot(h1, w2t_ref[...]) + be2_ref[...]

    ec = dot(ee_ref[...], wc1t_ref[...]) + bc1_ref[...]
    ec = ec * jax.nn.sigmoid(ec)
    enc = dot(ec, wc2t_ref[...]) + bc2_ref[...]

    p = dot_t(enc, wb0_ref[...])
    simT = dot_t(p, emb) + bb_ref[0, 0]

    logitsT = jnp.transpose(fused[:, D:D + E])
    combT = logitsT + 0.3 * simT
    m = jnp.max(combT, axis=0, keepdims=True)
    ex = jnp.exp(combT - m)
    scT = ex / jnp.sum(ex, axis=0, keepdims=True)
    scores_ref[...] = scT

    idx = jax.lax.broadcasted_iota(jnp.int32, scT.shape, 0)
    s1 = jnp.max(scT, axis=0, keepdims=True)
    i1 = jnp.min(jnp.where(scT == s1, idx, E), axis=0, keepdims=True)
    masked = jnp.where(idx == i1, -1.0, scT)
    s2 = jnp.max(masked, axis=0, keepdims=True)
    i2 = jnp.min(jnp.where(masked == s2, idx, E), axis=0, keepdims=True)

    t = jnp.exp(s2 - s1)
    p1 = 1.0 / (1.0 + t)
    io = jax.lax.broadcasted_iota(jnp.int32, (K, s1.shape[1]), 0)
    ts_ref[...] = jnp.where(io == 0, p1, t * p1)
    ti_ref[...] = jnp.where(io == 0, i1, i2)


@functools.partial(jax.jit, static_argnames=("interpret",))
def _router(xf, wcat, be1, w2t, be2, wc1t, bc1, wc2t, bc2, ee, wb0, bb2,
            interpret=False):
    grid = (N // TB,)
    full = lambda shape: pl.BlockSpec(shape, lambda i: (0,) * len(shape))
    return pl.pallas_call(
        _router_body,
        grid=grid,
        in_specs=[
            pl.BlockSpec((TB, H), lambda i: (i, 0)),
            full((H, 128)), full((1, D)), full((D, D)),
            full((1, D)), full((D, D)), full((1, D)), full((D, D)),
            full((1, D)), full((E, D)), full((D, D)), full((1, 1)),
        ],
        out_specs=[
            pl.BlockSpec((E, TB), lambda i: (0, i)),
            pl.BlockSpec((K, TB), lambda i: (0, i)),
            pl.BlockSpec((K, TB), lambda i: (0, i)),
        ],
        out_shape=[
            jax.ShapeDtypeStruct((E, N), jnp.float32),
            jax.ShapeDtypeStruct((K, N), jnp.float32),
            jax.ShapeDtypeStruct((K, N), jnp.int32),
        ],
        compiler_params=pltpu.CompilerParams(
            dimension_semantics=("parallel",)),
        interpret=interpret,
    )(xf, wcat, be1, w2t, be2, wc1t, bc1, wc2t, bc2, ee, wb0, bb2)


def kernel(x, gate_W, We1, be1, We2, be2, Wc1, bc1, Wc2, bc2,
           expert_embeddings, Wb, bb, interpret=False):
    xf = x.reshape(-1, H)
    wcat = jnp.concatenate(
        [We1.T, gate_W.T, jnp.zeros((H, 128 - D - E), jnp.float32)], axis=1)
    scoresT, tsT, tiT = _router(
        xf, wcat, be1.reshape(1, D), We2.T, be2.reshape(1, D),
        Wc1.T, bc1.reshape(1, D), Wc2.T, bc2.reshape(1, D),
        expert_embeddings, Wb[0], bb.reshape(1, 1), interpret=interpret)
    return (tsT.T, tiT.T, scoresT.T)
